# baseline (speedup 1.0000x reference)
import functools

import jax
import jax.numpy as jnp
from jax import lax
from jax.experimental import pallas as pl
from jax.experimental.pallas import tpu as pltpu
from jax.experimental.pallas import tpu_sc as plsc

N = 10000
E = 320000
D = 128
NP = 10240
NC, NS, L = 2, 16, 16
NW = NC * NS
EB = 128
NBAT2 = 2560
EPAD = NBAT2 * EB
PAD_NODE = 10200
CH = 16
NR = 10240
RPT = NR // NS

def _sc_mesh():
    return plsc.VectorSubcoreMesh(
        core_axis_name="c", subcore_axis_name="s", num_cores=NC, num_subcores=NS)

_f32 = jnp.float32


def _fill_1d(ref, n, value):
    def body(i, _):
        ref[pl.ds(i * 16, 16)] = jnp.full((16,), value, _f32)
        return 0
    lax.fori_loop(0, n // 16, body, 0)


def _zero_rows(ref, rows):
    def body(i, _):
        r = i // (D // 16)
        j = i % (D // 16)
        ref[r, pl.ds(j * 16, 16)] = jnp.zeros((16,), _f32)
        return 0
    lax.fori_loop(0, rows * (D // 16), body, 0)


DEG_NB = NBAT2 // NS


@functools.cache
def _degrees_kernel():
    return pl.kernel(
        _degrees_body,
        out_type=jax.ShapeDtypeStruct((NC, NP), _f32),
        mesh=_sc_mesh(),
        scratch_types=[
            pltpu.VMEM((DEG_NB, EB), jnp.int32),
            pltpu.VMEM((EB,), _f32),
            pltpu.VMEM((NP // NS,), _f32),
            pltpu.VMEM_SHARED((NP,), _f32),
            pltpu.SemaphoreType.DMA,
            pltpu.SemaphoreType.DMA,
            pltpu.SemaphoreType.DMA,
            pltpu.SemaphoreType.DMA,
        ],
    )


def _degrees(e3):
    return _degrees_kernel()(e3)


def _degrees_body(edge_ref, out_ref, idx_i, ones_v, zbuf_v, acc, s0, s1, s2, s3):
    c = lax.axis_index("c")
    s = lax.axis_index("s")
    ssem = [s0, s1, s2, s3]
    seg = NP // NS
    pltpu.sync_copy(edge_ref.at[c, pl.ds(s * DEG_NB, DEG_NB)], idx_i)
    _fill_1d(ones_v, EB, 1.0)
    _fill_1d(zbuf_v, seg, 0.0)
    pltpu.sync_copy(zbuf_v, acc.at[pl.ds(s * seg, seg)])
    plsc.subcore_barrier()

    def drain(b):
        pltpu.make_async_copy(edge_ref.at[0, 0], idx_i.at[0], ssem[b]).wait()

    def group(g, _):
        for b in range(4):
            t = g * 4 + b

            @pl.when(t >= 4)
            def _():
                drain(b)
            pltpu.async_copy(ones_v, acc.at[idx_i.at[t]], ssem[b], add=True)
        return 0
    lax.fori_loop(0, DEG_NB // 4, group, 0)
    for b in range(4):
        drain(b)
    plsc.subcore_barrier()
    pltpu.sync_copy(acc.at[pl.ds(s * seg, seg)], out_ref.at[c, pl.ds(s * seg, seg)])


def _seg_scratch():
    return [
        pltpu.VMEM((2, CH, EB), jnp.int32),
        pltpu.VMEM((2, CH, EB), jnp.int32),
        pltpu.VMEM((2, EB, D), _f32),
        pltpu.VMEM_SHARED((NR, D), _f32),
    ] + [pltpu.SemaphoreType.DMA] * 5


def _edge_pipeline(h_row_ref, edge_ref, out_ref, src_i, dst_i, rows_v, acc,
                   g0, g1, s0, s1, isem, c, s, first_bid, nb):
    gsem = (g0, g1)
    ssem = (s0, s1)
    nchunks = nb // CH
    pltpu.sync_copy(edge_ref.at[0, pl.ds(first_bid, CH)], src_i.at[0])
    pltpu.sync_copy(edge_ref.at[1, pl.ds(first_bid, CH)], dst_i.at[0])
    def zbody(i, _):
        r = i // (D // 16)
        j = i % (D // 16)
        rows_v[0, r, pl.ds(j * 16, 16)] = jnp.zeros((16,), _f32)
        return 0
    lax.fori_loop(0, EB * (D // 16), zbody, 0)
    for k in range(RPT // EB):
        pltpu.sync_copy(rows_v.at[0], acc.at[pl.ds(s * RPT + k * EB, EB)])
    pltpu.async_copy(h_row_ref.at[src_i.at[0, 0]], rows_v.at[0], gsem[0])
    plsc.subcore_barrier()

    def gwait(b):
        pltpu.make_async_copy(h_row_ref.at[pl.ds(0, EB)], rows_v.at[b],
                              gsem[b]).wait()

    def swait(b):
        pltpu.make_async_copy(h_row_ref.at[pl.ds(0, EB)], rows_v.at[b],
                              ssem[b]).wait()

    def chunk(k, _):
        p = k % 2
        for j in range(CH):
            b = j % 2
            gwait(b)
            pltpu.async_copy(rows_v.at[b], acc.at[dst_i.at[p, j]], ssem[b],
                             add=True)
            if j == 2:
                @pl.when(k < nchunks - 1)
                def _():
                    nxt = first_bid + (k + 1) * CH
                    pltpu.async_copy(edge_ref.at[0, pl.ds(nxt, CH)],
                                     src_i.at[1 - p], isem)
                    pltpu.async_copy(edge_ref.at[1, pl.ds(nxt, CH)],
                                     dst_i.at[1 - p], isem)
            if j < CH - 1:
                if j == 0:
                    @pl.when(k > 0)
                    def _():
                        swait(1 - b)
                else:
                    swait(1 - b)
                pltpu.async_copy(h_row_ref.at[src_i.at[p, j + 1]],
                                 rows_v.at[1 - b], gsem[1 - b])
            else:
                @pl.when(k < nchunks - 1)
                def _():
                    pltpu.make_async_copy(edge_ref.at[0, pl.ds(0, CH)],
                                          src_i.at[0], isem).wait()
                    pltpu.make_async_copy(edge_ref.at[0, pl.ds(0, CH)],
                                          dst_i.at[0], isem).wait()
                    swait(1 - b)
                    pltpu.async_copy(h_row_ref.at[src_i.at[1 - p, 0]],
                                     rows_v.at[1 - b], gsem[1 - b])
        return 0
    lax.fori_loop(0, nchunks, chunk, 0)
    swait(0)
    swait(1)
    plsc.subcore_barrier()
    pltpu.sync_copy(acc.at[pl.ds(s * RPT, RPT)], out_ref.at[c, pl.ds(s * RPT, RPT)])


SEG1_NB = NBAT2 // NW


@functools.cache
def _seg1_kernel():
    return pl.kernel(
        _seg1_body,
        out_type=jax.ShapeDtypeStruct((NC, NR, D), _f32),
        mesh=_sc_mesh(),
        scratch_types=_seg_scratch(),
    )


def _seg1(h, e3):
    return _seg1_kernel()(h, e3)


def _seg1_body(h_ref, edge_ref, out_ref, src_i, dst_i, rows_v, acc, *sems):
    c = lax.axis_index("c")
    s = lax.axis_index("s")
    wid = s * NC + c
    _edge_pipeline(h_ref, edge_ref, out_ref, src_i, dst_i, rows_v, acc,
                   *sems, c, s, wid * SEG1_NB, SEG1_NB)


SEG2_NB = NBAT2 // NS


@functools.cache
def _seg2_kernel():
    return pl.kernel(
        _seg2_body,
        out_type=jax.ShapeDtypeStruct((NC, NR, D), _f32),
        mesh=_sc_mesh(),
        scratch_types=_seg_scratch(),
    )


def _seg2(h2, e3):
    return _seg2_kernel()(h2, e3)


def _seg2_body(h2_ref, edge_ref, out_ref, src_i, dst_i, rows_v, acc, *sems):
    c = lax.axis_index("c")
    s = lax.axis_index("s")
    _edge_pipeline(h2_ref.at[c], edge_ref, out_ref, src_i, dst_i, rows_v, acc,
                   *sems, c, s, s * SEG2_NB, SEG2_NB)


RB = 1000
GRID = N // RB


def _mm_scale_body(x_ref, w_ref, deg_ref, o_ref):
    inv = lax.rsqrt(jnp.maximum(deg_ref[0], 1.0))
    o_ref[...] = jnp.dot(x_ref[...], w_ref[...],
                         preferred_element_type=_f32) * inv


def _mm_scale(x, W0, degs3):
    return pl.pallas_call(
        _mm_scale_body,
        grid=(GRID,),
        in_specs=[
            pl.BlockSpec((RB, D), lambda i: (i, 0)),
            pl.BlockSpec((D, D), lambda i: (0, 0)),
            pl.BlockSpec((NC, RB, 1), lambda i: (0, i, 0)),
        ],
        out_specs=pl.BlockSpec((RB, D), lambda i: (i, 0)),
        out_shape=jax.ShapeDtypeStruct((NR, D), _f32),
    )(x, W0, degs3)


def _mid_body(p_ref, deg_ref, b0_ref, w1_ref, o_ref):
    inv_out = lax.rsqrt(jnp.maximum(deg_ref[0], 1.0))
    inv_in = lax.rsqrt(jnp.maximum(deg_ref[1], 1.0))
    h1 = jnp.maximum((p_ref[0] + p_ref[1]) * inv_in + b0_ref[...], 0.0)
    hn = h1 * inv_out
    o_ref[0] = jnp.dot(hn, w1_ref[:, :D], preferred_element_type=_f32)
    o_ref[1] = jnp.dot(hn, w1_ref[:, D:], preferred_element_type=_f32)


def _mid(agg1, degs3, b0r, W1):
    return pl.pallas_call(
        _mid_body,
        grid=(GRID,),
        in_specs=[
            pl.BlockSpec((NC, RB, D), lambda i: (0, i, 0)),
            pl.BlockSpec((NC, RB, 1), lambda i: (0, i, 0)),
            pl.BlockSpec((1, D), lambda i: (0, 0)),
            pl.BlockSpec((D, 2 * D), lambda i: (0, 0)),
        ],
        out_specs=pl.BlockSpec((NC, RB, D), lambda i: (0, i, 0)),
        out_shape=jax.ShapeDtypeStruct((NC, NR, D), _f32),
    )(agg1, degs3, b0r, W1)


def _tail_body(p_ref, deg_ref, b1_ref, eps_ref, x_ref, we_ref, wd0_ref,
               bd0_ref, wd1_ref, bd1_ref, o_ref, acc):
    i = pl.program_id(0)

    @pl.when(i == 0)
    def _():
        acc[0] = 0.0
        acc[1] = 0.0

    inv_in = lax.rsqrt(jnp.maximum(deg_ref[1], 1.0))
    mu = p_ref[0] * inv_in + b1_ref[0]
    logvar = p_ref[1] * inv_in + b1_ref[1]
    z = mu + eps_ref[...] * jnp.exp(0.5 * logvar)
    rep = jnp.dot(z, we_ref[...], preferred_element_type=_f32)
    hdec = jnp.maximum(
        jnp.dot(rep, wd0_ref[...], preferred_element_type=_f32) + bd0_ref[...],
        0.0)
    logits = jnp.dot(hdec, wd1_ref[...], preferred_element_type=_f32) + bd1_ref[...]
    recon = jax.nn.sigmoid(logits)
    p = jnp.clip(recon, 1e-7, 1.0 - 1e-7)
    xb = x_ref[...]
    bce = -jnp.sum(xb * jnp.log(p) + (1.0 - xb) * jnp.log1p(-p))
    klin = jnp.sum(1.0 + logvar - mu * mu - jnp.exp(logvar))
    acc[0] += bce
    acc[1] += klin

    @pl.when(i == pl.num_programs(0) - 1)
    def _():
        o_ref[0, 0] = acc[0]
        o_ref[0, 1] = acc[1]


def _tail(agg2, degs3, b1r, eps, x, W_e2d, Wd0, bd0r, Wd1, bd1r):
    return pl.pallas_call(
        _tail_body,
        grid=(GRID,),
        in_specs=[
            pl.BlockSpec((NC, RB, D), lambda i: (0, i, 0)),
            pl.BlockSpec((NC, RB, 1), lambda i: (0, i, 0)),
            pl.BlockSpec((2, 1, D), lambda i: (0, 0, 0)),
            pl.BlockSpec((RB, D), lambda i: (i, 0)),
            pl.BlockSpec((RB, D), lambda i: (i, 0)),
            pl.BlockSpec((D, D), lambda i: (0, 0)),
            pl.BlockSpec((D, D), lambda i: (0, 0)),
            pl.BlockSpec((1, D), lambda i: (0, 0)),
            pl.BlockSpec((D, D), lambda i: (0, 0)),
            pl.BlockSpec((1, D), lambda i: (0, 0)),
        ],
        out_specs=pl.BlockSpec(memory_space=pltpu.MemorySpace.SMEM),
        out_shape=jax.ShapeDtypeStruct((1, 2), _f32),
        scratch_shapes=[pltpu.SMEM((2,), _f32)],
    )(agg2, degs3, b1r, eps, x, W_e2d, Wd0, bd0r, Wd1, bd1r)


def kernel(x, edge_index, W0, b0, W1, b1, W_e2d, Wd0, bd0, Wd1, bd1):
    e32 = edge_index.astype(jnp.int32)
    pad = jnp.full((2, EPAD - E), PAD_NODE, jnp.int32)
    e3 = jnp.concatenate([e32, pad], axis=1).reshape(2, NBAT2, EB)
    degs = _degrees(e3)
    degs3 = degs.reshape(NC, NP, 1)
    h0n = _mm_scale(x, W0, degs3)
    agg1 = _seg1(h0n, e3)
    h2 = _mid(agg1, degs3, b0.reshape(1, D), W1)
    agg2 = _seg2(h2, e3)
    eps = jax.random.normal(jax.random.key(42), (N, D), dtype=_f32)
    sums = _tail(agg2, degs3, b1.reshape(2, 1, D), eps, x,
                 W_e2d, Wd0, bd0.reshape(1, D), Wd1, bd1.reshape(1, D))
    bce_sum = sums[0, 0]
    klin = sums[0, 1]
    recon_loss = bce_sum / N
    kl_loss = -0.5 * klin / N
    err = -kl_loss
    Pk = 0.02 / (1.0 + jnp.exp(err)) + 0.5
    Ik = -0.001 * err
    Wk = jnp.maximum(Pk + Ik, 1e-6)
    return Wk * kl_loss + recon_loss
